# Initial kernel scaffold; baseline (speedup 1.0000x reference)
import jax
import jax.numpy as jnp
from jax.experimental import pallas as pl


def kernel(x, edge_weight, Wxz1, bxz1, Whz1, bhz1, Wxr1, bxr1, Whr1, bhr1, Wxh1, bxh1, Whh1, bhh1, Wxz2, bxz2, Whz2, bhz2, Wxr2, bxr2, Whr2, bhr2, Wxh2, bxh2, Whh2, bhh2, Wl, bl, edge_index):
    raise NotImplementedError("write your pallas kernel here")


# speedup vs baseline: 2.4421x; 2.4421x over previous
import jax
import jax.numpy as jnp
from jax.experimental import pallas as pl


def _power_weights(W):
    A0 = W[0] - W[2] + W[4]
    A1 = W[1] - 3.0 * W[3]
    A2 = 2.0 * W[2] - 8.0 * W[4]
    A3 = 4.0 * W[3]
    A4 = 8.0 * W[4]
    return [A0, A1, A2, A3, A4]


def _logsoftmax_kernel(x_ref, o_ref):
    x = x_ref[...]
    m = jnp.max(x, axis=-1, keepdims=True)
    e = jnp.exp(x - m)
    s = jnp.sum(e, axis=-1, keepdims=True)
    o_ref[...] = x - m - jnp.log(s)


def kernel(x, edge_weight, Wxz1, bxz1, Whz1, bhz1, Wxr1, bxr1, Whr1, bhr1,
           Wxh1, bxh1, Whh1, bhh1, Wxz2, bxz2, Whz2, bhz2, Wxr2, bxr2,
           Whr2, bhr2, Wxh2, bxh2, Whh2, bhh2, Wl, bl, edge_index):
    n = x.shape[0]
    row, col = edge_index[0], edge_index[1]
    ew = jnp.where(row == col, 0.0, edge_weight)
    deg = jax.ops.segment_sum(ew, row, num_segments=n)
    safe = jnp.where(deg > 0, deg, 1.0)
    dinv = jnp.where(deg > 0, 1.0 / jnp.sqrt(safe), 0.0)

    def spmm(v):
        return jax.ops.segment_sum(ew[:, None] * v[col], row, num_segments=n)

    def L_apply(v):
        return -dinv[:, None] * spmm(dinv[:, None] * v)

    def cheb_pair(X, Wz, bz_tot, Wh, bh_tot):
        Az = _power_weights(Wz)
        Ah = _power_weights(Wh)
        A = [jnp.concatenate([az, ah], axis=1) for az, ah in zip(Az, Ah)]
        Y = [X @ a for a in A]
        acc = Y[4]
        for j in (3, 2, 1, 0):
            acc = Y[j] + L_apply(acc)
        fz = Wz.shape[2]
        return acc[:, :fz] + bz_tot, acc[:, fz:] + bh_tot

    def layer(X, Wz, bxz, bhz, Wh, bxh, bhh):
        z, h = cheb_pair(X, Wz, bxz + bhz, Wh, bxh + bhh)
        return (1.0 - jax.nn.sigmoid(z)) * jnp.tanh(h)

    h1 = jax.nn.relu(layer(x, Wxz1, bxz1, bhz1, Wxh1, bxh1, bhh1))
    h2 = jax.nn.relu(layer(h1, Wxz2, bxz2, bhz2, Wxh2, bxh2, bhh2))
    logits = h2 @ Wl.T + bl

    lp = jnp.pad(logits, ((0, 240), (0, 0)))
    out = pl.pallas_call(
        _logsoftmax_kernel,
        out_shape=jax.ShapeDtypeStruct(lp.shape, lp.dtype),
        grid=(lp.shape[0] // 1280,),
        in_specs=[pl.BlockSpec((1280, 10), lambda i: (i, 0))],
        out_specs=pl.BlockSpec((1280, 10), lambda i: (i, 0)),
    )(lp)
    return out[:n]


# speedup vs baseline: 14.9280x; 6.1127x over previous
import functools

import jax
import jax.numpy as jnp
from jax import lax
from jax.experimental import pallas as pl
from jax.experimental.pallas import tpu as pltpu
from jax.experimental.pallas import tpu_sc as plsc

N = 10000
E = 320000
NP = 10240
EP = 327680
ROWS_PER_TILE = NP // 16
CHUNKS_PER_TILE = EP // 16 // 128
SUPER = 10
SUBS = 16


def _power_weights(W):
    A0 = W[0] - W[2] + W[4]
    A1 = W[1] - 3.0 * W[3]
    A2 = 2.0 * W[2] - 8.0 * W[4]
    A3 = 4.0 * W[3]
    A4 = 8.0 * W[4]
    return [A0, A1, A2, A3, A4]


def _zeros16():
    return jnp.zeros((16,), jnp.float32)


def _zero_2d(ref, nrows, fcols):
    def body(i, _):
        for b in range(fcols // 16):
            ref[i, pl.ds(16 * b, 16)] = _zeros16()
        return 0
    lax.fori_loop(0, nrows, body, 0)


def _deg_body(row2h, col2h, w2h, out, rowb, colb, wb, wmb, g16, zb, acc):
    c = lax.axis_index("c")
    s = lax.axis_index("s")
    r0 = s * ROWS_PER_TILE
    _zero_2d(g16, 128, 16)
    _zero_2d(zb, 128, 16)
    for ch in range(5):
        pltpu.sync_copy(zb, acc.at[pl.ds(r0 + 128 * ch, 128)])
    plsc.subcore_barrier()

    wid = c * 16 + s

    def super_body(sc, _):
        rrow0 = wid * (CHUNKS_PER_TILE // 2) + sc * SUBS
        pltpu.sync_copy(row2h.at[pl.ds(rrow0, SUBS)], rowb)
        pltpu.sync_copy(col2h.at[pl.ds(rrow0, SUBS)], colb)
        pltpu.sync_copy(w2h.at[pl.ds(rrow0, SUBS)], wb)

        def wmloop(i, _):
            for l in range(8):
                sl = pl.ds(16 * l, 16)
                rv = rowb[i, sl]
                cv = colb[i, sl]
                wv = wb[i, sl]
                wmb[pl.ds(128 * i + 16 * l, 16)] = jnp.where(rv == cv, 0.0, wv)
            return 0
        lax.fori_loop(0, SUBS, wmloop, 0)

        def sub_body(k, _):
            def grp(g, _):
                wv = wmb[pl.ds(128 * k + 16 * g, 16)]
                for l in range(16):
                    g16[16 * g + l, pl.ds(0, 16)] = jnp.full((16,), wv[l])
                return 0
            lax.fori_loop(0, 8, grp, 0)
            pltpu.sync_copy(g16, acc.at[rowb.at[k]], add=True)
            return 0
        lax.fori_loop(0, SUBS, sub_body, 0)
        return 0
    lax.fori_loop(0, SUPER // 2, super_body, 0)

    plsc.subcore_barrier()
    for ch in range(5):
        r = r0 + 128 * ch
        pltpu.sync_copy(acc.at[pl.ds(r, 128)], out.at[c].at[pl.ds(r, 128)])


def _deg_call(row2, col2, w2):
    mesh = plsc.VectorSubcoreMesh(core_axis_name="c", subcore_axis_name="s")
    return pl.kernel(
        _deg_body,
        out_type=jax.ShapeDtypeStruct((2, NP, 16), jnp.float32),
        mesh=mesh,
        compiler_params=pltpu.CompilerParams(use_tc_tiling_on_sc=False),
        scratch_types=[
            pltpu.VMEM((SUBS, 128), jnp.int32),
            pltpu.VMEM((SUBS, 128), jnp.int32),
            pltpu.VMEM((SUBS, 128), jnp.float32),
            pltpu.VMEM((2048,), jnp.float32),
            pltpu.VMEM((128, 16), jnp.float32),
            pltpu.VMEM((128, 16), jnp.float32),
            pltpu.VMEM_SHARED((NP, 16), jnp.float32),
        ],
    )(row2, col2, w2)


def _layer_body(Fh, ys, d2x, row2h, col2h, w2h, p_out, u_scr,
                rowb, colb, wb, wmb, g0, g1, yb, pb, db, ub, zb,
                acc, sg0, sg1):
    c = lax.axis_index("c")
    s = lax.axis_index("s")
    r0 = s * ROWS_PER_TILE
    nb = Fh // 16
    _zero_2d(zb, 128, Fh)
    for ch in range(5):
        pltpu.sync_copy(zb, acc.at[pl.ds(r0 + 128 * ch, 128)])
    plsc.subcore_barrier()

    def multiply(G, wm_base):
        def grp(g, _):
            wv = wmb[pl.ds(wm_base + 16 * g, 16)]
            for l in range(16):
                e = 16 * g + l
                for b in range(nb):
                    sl = pl.ds(16 * b, 16)
                    G[e, sl] = G[e, sl] * wv[l]
            return 0
        lax.fori_loop(0, 8, grp, 0)

    for step in range(4):
        src = ys.at[c, 4] if step == 0 else u_scr.at[c]

        def super_body(sc, _):
            rrow0 = s * CHUNKS_PER_TILE + sc * SUBS
            pltpu.sync_copy(row2h.at[pl.ds(rrow0, SUBS)], rowb)
            pltpu.sync_copy(col2h.at[pl.ds(rrow0, SUBS)], colb)
            pltpu.sync_copy(w2h.at[pl.ds(rrow0, SUBS)], wb)

            def wmloop(i, _):
                for l in range(8):
                    sl = pl.ds(16 * l, 16)
                    rv = rowb[i, sl]
                    cv = colb[i, sl]
                    wv = wb[i, sl]
                    wmb[pl.ds(128 * i + 16 * l, 16)] = jnp.where(
                        rv == cv, 0.0, wv)
                return 0
            lax.fori_loop(0, SUBS, wmloop, 0)

            def pair(k, _):
                h0 = pltpu.async_copy(src.at[colb.at[2 * k]], g0, sg0)
                h1 = pltpu.async_copy(src.at[colb.at[2 * k + 1]], g1, sg1)
                h0.wait()
                multiply(g0, 128 * (2 * k))
                pltpu.sync_copy(g0, acc.at[rowb.at[2 * k]], add=True)
                h1.wait()
                multiply(g1, 128 * (2 * k + 1))
                pltpu.sync_copy(g1, acc.at[rowb.at[2 * k + 1]], add=True)
                return 0
            lax.fori_loop(0, SUBS // 2, pair, 0)
            return 0
        lax.fori_loop(0, SUPER, super_body, 0)
        plsc.subcore_barrier()

        if step < 3:
            j = 3 - step

            def glue(chn, _):
                r = r0 + 128 * chn
                pltpu.sync_copy(acc.at[pl.ds(r, 128)], pb)
                pltpu.sync_copy(zb, acc.at[pl.ds(r, 128)])
                pltpu.sync_copy(ys.at[c, j].at[pl.ds(r, 128)], yb)
                pltpu.sync_copy(d2x.at[pl.ds(r, 128)], db)

                def rowfn(i, _):
                    for b in range(nb):
                        sl = pl.ds(16 * b, 16)
                        ub[i, sl] = yb[i, sl] - db[i, sl] * pb[i, sl]
                    return 0
                lax.fori_loop(0, 128, rowfn, 0)
                pltpu.sync_copy(ub, u_scr.at[c].at[pl.ds(r, 128)])
                return 0
            lax.fori_loop(0, 5, glue, 0)
            plsc.subcore_barrier()
        else:
            for ch in range(5):
                r = r0 + 128 * ch
                pltpu.sync_copy(acc.at[pl.ds(r, 128)],
                                p_out.at[c].at[pl.ds(r, 128)])


def _layer_call(Fh, ys, d2x, row2, col2, w2):
    mesh = plsc.VectorSubcoreMesh(core_axis_name="c", subcore_axis_name="s")
    fb = lambda shape: pltpu.VMEM(shape, jnp.float32)
    return pl.kernel(
        functools.partial(_layer_body, Fh),
        out_type=(jax.ShapeDtypeStruct((2, NP, Fh), jnp.float32),
                  jax.ShapeDtypeStruct((2, NP, Fh), jnp.float32)),
        mesh=mesh,
        compiler_params=pltpu.CompilerParams(use_tc_tiling_on_sc=False),
        scratch_types=[
            pltpu.VMEM((SUBS, 128), jnp.int32),
            pltpu.VMEM((SUBS, 128), jnp.int32),
            pltpu.VMEM((SUBS, 128), jnp.float32),
            pltpu.VMEM((2048,), jnp.float32),
            fb((128, Fh)), fb((128, Fh)),
            fb((128, Fh)), fb((128, Fh)), fb((128, Fh)), fb((128, Fh)),
            fb((128, Fh)),
            pltpu.VMEM_SHARED((NP, Fh), jnp.float32),
            pltpu.SemaphoreType.DMA,
            pltpu.SemaphoreType.DMA,
        ],
    )(ys, d2x, row2, col2, w2)


def _prep_body(x_ref, a1_ref, deg_ref, ys_ref, d232_ref, d216_ref, dinv_ref):
    deg = deg_ref[0, :, 0] + deg_ref[1, :, 0]
    safe = jnp.where(deg > 0, deg, 1.0)
    dinv = jnp.where(deg > 0, lax.rsqrt(safe), 0.0)
    d2 = dinv * dinv
    y = jnp.dot(x_ref[...], a1_ref[...], preferred_element_type=jnp.float32)
    for c in range(2):
        for j in range(5):
            blk = y[:, c * 160 + j * 32:c * 160 + (j + 1) * 32]
            if j > 0:
                blk = dinv[:, None] * blk
            ys_ref[c, j, :, :] = blk
    d232_ref[...] = jnp.broadcast_to(d2[:, None], d232_ref.shape)
    d216_ref[...] = jnp.broadcast_to(d2[:, None], d216_ref.shape)
    dinv_ref[...] = jnp.broadcast_to(dinv[:, None], dinv_ref.shape)


def _prep_call(x_pad, a1, deg16):
    R = 512
    grid = (NP // R,)
    return pl.pallas_call(
        _prep_body,
        grid=grid,
        in_specs=[
            pl.BlockSpec((R, 128), lambda i: (i, 0)),
            pl.BlockSpec((128, 320), lambda i: (0, 0)),
            pl.BlockSpec((2, R, 16), lambda i: (0, i, 0)),
        ],
        out_specs=[
            pl.BlockSpec((2, 5, R, 32), lambda i: (0, 0, i, 0)),
            pl.BlockSpec((R, 32), lambda i: (i, 0)),
            pl.BlockSpec((R, 16), lambda i: (i, 0)),
            pl.BlockSpec((R, 8), lambda i: (i, 0)),
        ],
        out_shape=[
            jax.ShapeDtypeStruct((2, 5, NP, 32), jnp.float32),
            jax.ShapeDtypeStruct((NP, 32), jnp.float32),
            jax.ShapeDtypeStruct((NP, 16), jnp.float32),
            jax.ShapeDtypeStruct((NP, 8), jnp.float32),
        ],
    )(x_pad, a1, deg16)


def _mid_body(p_ref, y0_ref, dinv_ref, a2_ref, bz_ref, bh_ref, ys2_ref):
    dinv = dinv_ref[:, 0]
    z = y0_ref[0, 0] - dinv[:, None] * p_ref[0] + bz_ref[...]
    h = y0_ref[1, 0] - dinv[:, None] * p_ref[1] + bh_ref[...]
    h1 = jax.nn.relu((1.0 - jax.nn.sigmoid(z)) * jnp.tanh(h))
    y2 = jnp.dot(h1, a2_ref[...], preferred_element_type=jnp.float32)
    for c in range(2):
        for j in range(5):
            blk = y2[:, c * 80 + j * 16:c * 80 + (j + 1) * 16]
            if j > 0:
                blk = dinv[:, None] * blk
            ys2_ref[c, j, :, :] = blk


def _mid_call(p1, ys1, dinv8, a2, bz, bh):
    R = 512
    return pl.pallas_call(
        _mid_body,
        grid=(NP // R,),
        in_specs=[
            pl.BlockSpec((2, R, 32), lambda i: (0, i, 0)),
            pl.BlockSpec((2, 1, R, 32), lambda i: (0, 0, i, 0)),
            pl.BlockSpec((R, 8), lambda i: (i, 0)),
            pl.BlockSpec((32, 160), lambda i: (0, 0)),
            pl.BlockSpec((1, 32), lambda i: (0, 0)),
            pl.BlockSpec((1, 32), lambda i: (0, 0)),
        ],
        out_specs=pl.BlockSpec((2, 5, R, 16), lambda i: (0, 0, i, 0)),
        out_shape=jax.ShapeDtypeStruct((2, 5, NP, 16), jnp.float32),
    )(p1, ys1, dinv8, a2, bz, bh)


def _final_body(p_ref, y0_ref, dinv_ref, wl_ref, bl_ref, bz_ref, bh_ref,
                out_ref):
    dinv = dinv_ref[:, 0]
    z = y0_ref[0, 0] - dinv[:, None] * p_ref[0] + bz_ref[...]
    h = y0_ref[1, 0] - dinv[:, None] * p_ref[1] + bh_ref[...]
    h2 = jax.nn.relu((1.0 - jax.nn.sigmoid(z)) * jnp.tanh(h))
    logits = jnp.dot(h2, wl_ref[...],
                     preferred_element_type=jnp.float32) + bl_ref[...]
    m = jnp.max(logits, axis=-1, keepdims=True)
    e = jnp.exp(logits - m)
    ssum = jnp.sum(e, axis=-1, keepdims=True)
    out_ref[...] = logits - m - jnp.log(ssum)


def _final_call(p2, ys2, dinv8, wlp, blp, bz, bh):
    R = 512
    return pl.pallas_call(
        _final_body,
        grid=(NP // R,),
        in_specs=[
            pl.BlockSpec((2, R, 16), lambda i: (0, i, 0)),
            pl.BlockSpec((2, 1, R, 16), lambda i: (0, 0, i, 0)),
            pl.BlockSpec((R, 8), lambda i: (i, 0)),
            pl.BlockSpec((16, 128), lambda i: (0, 0)),
            pl.BlockSpec((1, 128), lambda i: (0, 0)),
            pl.BlockSpec((1, 16), lambda i: (0, 0)),
            pl.BlockSpec((1, 16), lambda i: (0, 0)),
        ],
        out_specs=pl.BlockSpec((R, 128), lambda i: (i, 0)),
        out_shape=jax.ShapeDtypeStruct((NP, 128), jnp.float32),
    )(p2, ys2, dinv8, wlp, blp, bz, bh)


def kernel(x, edge_weight, Wxz1, bxz1, Whz1, bhz1, Wxr1, bxr1, Whr1, bhr1,
           Wxh1, bxh1, Whh1, bhh1, Wxz2, bxz2, Whz2, bhz2, Wxr2, bxr2,
           Whr2, bhr2, Wxh2, bxh2, Whh2, bhh2, Wl, bl, edge_index):
    row = edge_index[0]
    col = edge_index[1]
    row2 = jnp.concatenate(
        [row, jnp.zeros((EP - E,), jnp.int32)]).reshape(EP // 128, 128)
    col2 = jnp.concatenate(
        [col, jnp.zeros((EP - E,), jnp.int32)]).reshape(EP // 128, 128)
    w2 = jnp.concatenate(
        [edge_weight, jnp.zeros((EP - E,), jnp.float32)]).reshape(
            EP // 128, 128)
    x_pad = jnp.pad(x, ((0, NP - N), (0, 0)))

    az1 = _power_weights(Wxz1)
    ah1 = _power_weights(Wxh1)
    a1 = jnp.concatenate([a for pair in (az1, ah1) for a in pair], axis=1)
    az2 = _power_weights(Wxz2)
    ah2 = _power_weights(Wxh2)
    a2 = jnp.concatenate([a for pair in (az2, ah2) for a in pair], axis=1)
    bz1 = (bxz1 + bhz1).reshape(1, 32)
    bh1 = (bxh1 + bhh1).reshape(1, 32)
    bz2 = (bxz2 + bhz2).reshape(1, 16)
    bh2 = (bxh2 + bhh2).reshape(1, 16)
    wlp = jnp.pad(Wl.T, ((0, 0), (0, 118)))
    blp = jnp.pad(bl, (0, 118), constant_values=-1e30).reshape(1, 128)

    deg16 = _deg_call(row2, col2, w2)
    ys1, d232, d216, dinv8 = _prep_call(x_pad, a1, deg16)
    p1, _ = _layer_call(32, ys1, d232, row2, col2, w2)
    ys2 = _mid_call(p1, ys1, dinv8, a2, bz1, bh1)
    p2, _ = _layer_call(16, ys2, d216, row2, col2, w2)
    outp = _final_call(p2, ys2, dinv8, wlp, blp, bz2, bh2)
    return outp[:N, :10]


# speedup vs baseline: 17.8365x; 1.1948x over previous
import functools

import jax
import jax.numpy as jnp
from jax import lax
from jax.experimental import pallas as pl
from jax.experimental.pallas import tpu as pltpu
from jax.experimental.pallas import tpu_sc as plsc

N = 10000
E = 320000
NP = 10240
EP = 327680
ROWS_PER_TILE = NP // 16
CHUNKS_PER_TILE = EP // 16 // 128
SUPER = 10
SUBS = 16


def _power_weights(W):
    A0 = W[0] - W[2] + W[4]
    A1 = W[1] - 3.0 * W[3]
    A2 = 2.0 * W[2] - 8.0 * W[4]
    A3 = 4.0 * W[3]
    A4 = 8.0 * W[4]
    return [A0, A1, A2, A3, A4]


def _zeros16():
    return jnp.zeros((16,), jnp.float32)


def _zero_2d(ref, nrows, fcols):
    def body(i, _):
        for b in range(fcols // 16):
            ref[i, pl.ds(16 * b, 16)] = _zeros16()
        return 0
    lax.fori_loop(0, nrows, body, 0)


def _deg_body(row2h, col2h, w2h, out, rowb, colb, wb, wmb, g16, zb, acc):
    c = lax.axis_index("c")
    s = lax.axis_index("s")
    r0 = s * ROWS_PER_TILE
    _zero_2d(g16, 128, 16)
    _zero_2d(zb, 128, 16)
    for ch in range(5):
        pltpu.sync_copy(zb, acc.at[pl.ds(r0 + 128 * ch, 128)])
    plsc.subcore_barrier()

    wid = c * 16 + s

    def super_body(sc, _):
        rrow0 = wid * (CHUNKS_PER_TILE // 2) + sc * SUBS
        pltpu.sync_copy(row2h.at[pl.ds(rrow0, SUBS)], rowb)
        pltpu.sync_copy(col2h.at[pl.ds(rrow0, SUBS)], colb)
        pltpu.sync_copy(w2h.at[pl.ds(rrow0, SUBS)], wb)

        def wmloop(i, _):
            for l in range(8):
                sl = pl.ds(16 * l, 16)
                rv = rowb[i, sl]
                cv = colb[i, sl]
                wv = wb[i, sl]
                wmb[pl.ds(128 * i + 16 * l, 16)] = jnp.where(rv == cv, 0.0, wv)
            return 0
        lax.fori_loop(0, SUBS, wmloop, 0)

        def sub_body(k, _):
            def grp(g, _):
                wv = wmb[pl.ds(128 * k + 16 * g, 16)]
                for l in range(16):
                    g16[16 * g + l, pl.ds(0, 16)] = jnp.full((16,), wv[l])
                return 0
            lax.fori_loop(0, 8, grp, 0)
            pltpu.sync_copy(g16, acc.at[rowb.at[k]], add=True)
            return 0
        lax.fori_loop(0, SUBS, sub_body, 0)
        return 0
    lax.fori_loop(0, SUPER // 2, super_body, 0)

    plsc.subcore_barrier()
    for ch in range(5):
        r = r0 + 128 * ch
        pltpu.sync_copy(acc.at[pl.ds(r, 128)], out.at[c].at[pl.ds(r, 128)])


def _deg_call(row2, col2, w2):
    mesh = plsc.VectorSubcoreMesh(core_axis_name="c", subcore_axis_name="s")
    return pl.kernel(
        _deg_body,
        out_type=jax.ShapeDtypeStruct((2, NP, 16), jnp.float32),
        mesh=mesh,
        compiler_params=pltpu.CompilerParams(use_tc_tiling_on_sc=False),
        scratch_types=[
            pltpu.VMEM((SUBS, 128), jnp.int32),
            pltpu.VMEM((SUBS, 128), jnp.int32),
            pltpu.VMEM((SUBS, 128), jnp.float32),
            pltpu.VMEM((2048,), jnp.float32),
            pltpu.VMEM((128, 16), jnp.float32),
            pltpu.VMEM((128, 16), jnp.float32),
            pltpu.VMEM_SHARED((NP, 16), jnp.float32),
        ],
    )(row2, col2, w2)


def _layer_body(Fh, ys, d2x, row2h, col2h, w2h, p_out, u_scr,
                rowb, colb, wb, wmb, g0, g1, g2, g3, yb, pb, db, ub, zb,
                acc, sg0, sg1, sg2, sg3, ss0, ss1, ss2, ss3):
    gbufs = (g0, g1, g2, g3)
    sgs = (sg0, sg1, sg2, sg3)
    sss = (ss0, ss1, ss2, ss3)
    c = lax.axis_index("c")
    s = lax.axis_index("s")
    r0 = s * ROWS_PER_TILE
    nb = Fh // 16
    _zero_2d(zb, 128, Fh)
    for ch in range(5):
        pltpu.sync_copy(zb, acc.at[pl.ds(r0 + 128 * ch, 128)])
    plsc.subcore_barrier()

    def multiply(G, wm_base):
        def grp(g, _):
            wv = wmb[pl.ds(wm_base + 16 * g, 16)]
            for l in range(16):
                e = 16 * g + l
                for b in range(nb):
                    sl = pl.ds(16 * b, 16)
                    G[e, sl] = G[e, sl] * wv[l]
            return 0
        lax.fori_loop(0, 8, grp, 0)

    for step in range(4):
        src = ys.at[c, 4] if step == 0 else u_scr.at[c]

        def super_body(sc, _):
            rrow0 = s * CHUNKS_PER_TILE + sc * SUBS
            pltpu.sync_copy(row2h.at[pl.ds(rrow0, SUBS)], rowb)
            pltpu.sync_copy(col2h.at[pl.ds(rrow0, SUBS)], colb)
            pltpu.sync_copy(w2h.at[pl.ds(rrow0, SUBS)], wb)

            def wmloop(i, _):
                for l in range(8):
                    sl = pl.ds(16 * l, 16)
                    rv = rowb[i, sl]
                    cv = colb[i, sl]
                    wv = wb[i, sl]
                    wmb[pl.ds(128 * i + 16 * l, 16)] = jnp.where(
                        rv == cv, 0.0, wv)
                return 0
            lax.fori_loop(0, SUBS, wmloop, 0)

            def quad(q, _):
                base = 4 * q
                ghs = [pltpu.async_copy(src.at[colb.at[base + b]],
                                        gbufs[b], sgs[b]) for b in range(4)]
                shs = []
                for b in range(4):
                    ghs[b].wait()
                    multiply(gbufs[b], 128 * (base + b))
                    shs.append(pltpu.async_copy(
                        gbufs[b], acc.at[rowb.at[base + b]], sss[b],
                        add=True))
                for h in shs:
                    h.wait()
                return 0
            lax.fori_loop(0, SUBS // 4, quad, 0)
            return 0
        lax.fori_loop(0, SUPER, super_body, 0)
        plsc.subcore_barrier()

        if step < 3:
            j = 3 - step

            def glue(chn, _):
                r = r0 + 128 * chn
                pltpu.sync_copy(acc.at[pl.ds(r, 128)], pb)
                pltpu.sync_copy(zb, acc.at[pl.ds(r, 128)])
                pltpu.sync_copy(ys.at[c, j].at[pl.ds(r, 128)], yb)
                pltpu.sync_copy(d2x.at[pl.ds(r, 128)], db)

                def rowfn(i, _):
                    for b in range(nb):
                        sl = pl.ds(16 * b, 16)
                        ub[i, sl] = yb[i, sl] - db[i, sl] * pb[i, sl]
                    return 0
                lax.fori_loop(0, 128, rowfn, 0)
                pltpu.sync_copy(ub, u_scr.at[c].at[pl.ds(r, 128)])
                return 0
            lax.fori_loop(0, 5, glue, 0)
            plsc.subcore_barrier()
        else:
            for ch in range(5):
                r = r0 + 128 * ch
                pltpu.sync_copy(acc.at[pl.ds(r, 128)],
                                p_out.at[c].at[pl.ds(r, 128)])


def _layer_call(Fh, ys, d2x, row2, col2, w2):
    mesh = plsc.VectorSubcoreMesh(core_axis_name="c", subcore_axis_name="s")
    fb = lambda shape: pltpu.VMEM(shape, jnp.float32)
    return pl.kernel(
        functools.partial(_layer_body, Fh),
        out_type=(jax.ShapeDtypeStruct((2, NP, Fh), jnp.float32),
                  jax.ShapeDtypeStruct((2, NP, Fh), jnp.float32)),
        mesh=mesh,
        compiler_params=pltpu.CompilerParams(use_tc_tiling_on_sc=False),
        scratch_types=[
            pltpu.VMEM((SUBS, 128), jnp.int32),
            pltpu.VMEM((SUBS, 128), jnp.int32),
            pltpu.VMEM((SUBS, 128), jnp.float32),
            pltpu.VMEM((2048,), jnp.float32),
            fb((128, Fh)), fb((128, Fh)),
            fb((128, Fh)), fb((128, Fh)),
            fb((128, Fh)), fb((128, Fh)), fb((128, Fh)), fb((128, Fh)),
            fb((128, Fh)),
            pltpu.VMEM_SHARED((NP, Fh), jnp.float32),
        ] + [pltpu.SemaphoreType.DMA] * 8,
    )(ys, d2x, row2, col2, w2)


def _prep_body(x_ref, a1_ref, deg_ref, ys_ref, d232_ref, d216_ref, dinv_ref):
    deg = deg_ref[0, :, 0] + deg_ref[1, :, 0]
    safe = jnp.where(deg > 0, deg, 1.0)
    dinv = jnp.where(deg > 0, lax.rsqrt(safe), 0.0)
    d2 = dinv * dinv
    y = jnp.dot(x_ref[...], a1_ref[...], preferred_element_type=jnp.float32)
    for c in range(2):
        for j in range(5):
            blk = y[:, c * 160 + j * 32:c * 160 + (j + 1) * 32]
            if j > 0:
                blk = dinv[:, None] * blk
            ys_ref[c, j, :, :] = blk
    d232_ref[...] = jnp.broadcast_to(d2[:, None], d232_ref.shape)
    d216_ref[...] = jnp.broadcast_to(d2[:, None], d216_ref.shape)
    dinv_ref[...] = jnp.broadcast_to(dinv[:, None], dinv_ref.shape)


def _prep_call(x_pad, a1, deg16):
    R = 512
    grid = (NP // R,)
    return pl.pallas_call(
        _prep_body,
        grid=grid,
        in_specs=[
            pl.BlockSpec((R, 128), lambda i: (i, 0)),
            pl.BlockSpec((128, 320), lambda i: (0, 0)),
            pl.BlockSpec((2, R, 16), lambda i: (0, i, 0)),
        ],
        out_specs=[
            pl.BlockSpec((2, 5, R, 32), lambda i: (0, 0, i, 0)),
            pl.BlockSpec((R, 32), lambda i: (i, 0)),
            pl.BlockSpec((R, 16), lambda i: (i, 0)),
            pl.BlockSpec((R, 8), lambda i: (i, 0)),
        ],
        out_shape=[
            jax.ShapeDtypeStruct((2, 5, NP, 32), jnp.float32),
            jax.ShapeDtypeStruct((NP, 32), jnp.float32),
            jax.ShapeDtypeStruct((NP, 16), jnp.float32),
            jax.ShapeDtypeStruct((NP, 8), jnp.float32),
        ],
    )(x_pad, a1, deg16)


def _mid_body(p_ref, y0_ref, dinv_ref, a2_ref, bz_ref, bh_ref, ys2_ref):
    dinv = dinv_ref[:, 0]
    z = y0_ref[0, 0] - dinv[:, None] * p_ref[0] + bz_ref[...]
    h = y0_ref[1, 0] - dinv[:, None] * p_ref[1] + bh_ref[...]
    h1 = jax.nn.relu((1.0 - jax.nn.sigmoid(z)) * jnp.tanh(h))
    y2 = jnp.dot(h1, a2_ref[...], preferred_element_type=jnp.float32)
    for c in range(2):
        for j in range(5):
            blk = y2[:, c * 80 + j * 16:c * 80 + (j + 1) * 16]
            if j > 0:
                blk = dinv[:, None] * blk
            ys2_ref[c, j, :, :] = blk


def _mid_call(p1, ys1, dinv8, a2, bz, bh):
    R = 512
    return pl.pallas_call(
        _mid_body,
        grid=(NP // R,),
        in_specs=[
            pl.BlockSpec((2, R, 32), lambda i: (0, i, 0)),
            pl.BlockSpec((2, 1, R, 32), lambda i: (0, 0, i, 0)),
            pl.BlockSpec((R, 8), lambda i: (i, 0)),
            pl.BlockSpec((32, 160), lambda i: (0, 0)),
            pl.BlockSpec((1, 32), lambda i: (0, 0)),
            pl.BlockSpec((1, 32), lambda i: (0, 0)),
        ],
        out_specs=pl.BlockSpec((2, 5, R, 16), lambda i: (0, 0, i, 0)),
        out_shape=jax.ShapeDtypeStruct((2, 5, NP, 16), jnp.float32),
    )(p1, ys1, dinv8, a2, bz, bh)


def _final_body(p_ref, y0_ref, dinv_ref, wl_ref, bl_ref, bz_ref, bh_ref,
                out_ref):
    dinv = dinv_ref[:, 0]
    z = y0_ref[0, 0] - dinv[:, None] * p_ref[0] + bz_ref[...]
    h = y0_ref[1, 0] - dinv[:, None] * p_ref[1] + bh_ref[...]
    h2 = jax.nn.relu((1.0 - jax.nn.sigmoid(z)) * jnp.tanh(h))
    logits = jnp.dot(h2, wl_ref[...],
                     preferred_element_type=jnp.float32) + bl_ref[...]
    m = jnp.max(logits, axis=-1, keepdims=True)
    e = jnp.exp(logits - m)
    ssum = jnp.sum(e, axis=-1, keepdims=True)
    out_ref[...] = logits - m - jnp.log(ssum)


def _final_call(p2, ys2, dinv8, wlp, blp, bz, bh):
    R = 512
    return pl.pallas_call(
        _final_body,
        grid=(NP // R,),
        in_specs=[
            pl.BlockSpec((2, R, 16), lambda i: (0, i, 0)),
            pl.BlockSpec((2, 1, R, 16), lambda i: (0, 0, i, 0)),
            pl.BlockSpec((R, 8), lambda i: (i, 0)),
            pl.BlockSpec((16, 128), lambda i: (0, 0)),
            pl.BlockSpec((1, 128), lambda i: (0, 0)),
            pl.BlockSpec((1, 16), lambda i: (0, 0)),
            pl.BlockSpec((1, 16), lambda i: (0, 0)),
        ],
        out_specs=pl.BlockSpec((R, 128), lambda i: (i, 0)),
        out_shape=jax.ShapeDtypeStruct((NP, 128), jnp.float32),
    )(p2, ys2, dinv8, wlp, blp, bz, bh)


def kernel(x, edge_weight, Wxz1, bxz1, Whz1, bhz1, Wxr1, bxr1, Whr1, bhr1,
           Wxh1, bxh1, Whh1, bhh1, Wxz2, bxz2, Whz2, bhz2, Wxr2, bxr2,
           Whr2, bhr2, Wxh2, bxh2, Whh2, bhh2, Wl, bl, edge_index):
    row = edge_index[0]
    col = edge_index[1]
    row2 = jnp.concatenate(
        [row, jnp.zeros((EP - E,), jnp.int32)]).reshape(EP // 128, 128)
    col2 = jnp.concatenate(
        [col, jnp.zeros((EP - E,), jnp.int32)]).reshape(EP // 128, 128)
    w2 = jnp.concatenate(
        [edge_weight, jnp.zeros((EP - E,), jnp.float32)]).reshape(
            EP // 128, 128)
    x_pad = jnp.pad(x, ((0, NP - N), (0, 0)))

    az1 = _power_weights(Wxz1)
    ah1 = _power_weights(Wxh1)
    a1 = jnp.concatenate([a for pair in (az1, ah1) for a in pair], axis=1)
    az2 = _power_weights(Wxz2)
    ah2 = _power_weights(Wxh2)
    a2 = jnp.concatenate([a for pair in (az2, ah2) for a in pair], axis=1)
    bz1 = (bxz1 + bhz1).reshape(1, 32)
    bh1 = (bxh1 + bhh1).reshape(1, 32)
    bz2 = (bxz2 + bhz2).reshape(1, 16)
    bh2 = (bxh2 + bhh2).reshape(1, 16)
    wlp = jnp.pad(Wl.T, ((0, 0), (0, 118)))
    blp = jnp.pad(bl, (0, 118), constant_values=-1e30).reshape(1, 128)

    deg16 = _deg_call(row2, col2, w2)
    ys1, d232, d216, dinv8 = _prep_call(x_pad, a1, deg16)
    p1, _ = _layer_call(32, ys1, d232, row2, col2, w2)
    ys2 = _mid_call(p1, ys1, dinv8, a2, bz1, bh1)
    p2, _ = _layer_call(16, ys2, d216, row2, col2, w2)
    outp = _final_call(p2, ys2, dinv8, wlp, blp, bz2, bh2)
    return outp[:N, :10]


# speedup vs baseline: 17.8784x; 1.0023x over previous
import functools

import jax
import jax.numpy as jnp
from jax import lax
from jax.experimental import pallas as pl
from jax.experimental.pallas import tpu as pltpu
from jax.experimental.pallas import tpu_sc as plsc

N = 10000
E = 320000
NP = 10240
EP = 327680
ROWS_PER_TILE = NP // 16
CHUNKS_PER_TILE = EP // 16 // 128
SUPER = 10
SUBS = 16


def _power_weights(W):
    A0 = W[0] - W[2] + W[4]
    A1 = W[1] - 3.0 * W[3]
    A2 = 2.0 * W[2] - 8.0 * W[4]
    A3 = 4.0 * W[3]
    A4 = 8.0 * W[4]
    return [A0, A1, A2, A3, A4]


def _zeros16():
    return jnp.zeros((16,), jnp.float32)


def _zero_2d(ref, nrows, fcols):
    def body(i, _):
        for b in range(fcols // 16):
            ref[i, pl.ds(16 * b, 16)] = _zeros16()
        return 0
    lax.fori_loop(0, nrows, body, 0)


def _deg_body(row2h, col2h, w2h, out, rowb, colb, wb, wmb, g16, zb, acc):
    c = lax.axis_index("c")
    s = lax.axis_index("s")
    r0 = s * ROWS_PER_TILE
    _zero_2d(g16, 128, 16)
    _zero_2d(zb, 128, 16)
    for ch in range(5):
        pltpu.sync_copy(zb, acc.at[pl.ds(r0 + 128 * ch, 128)])
    plsc.subcore_barrier()

    wid = c * 16 + s

    def super_body(sc, _):
        rrow0 = wid * (CHUNKS_PER_TILE // 2) + sc * SUBS
        pltpu.sync_copy(row2h.at[pl.ds(rrow0, SUBS)], rowb)
        pltpu.sync_copy(col2h.at[pl.ds(rrow0, SUBS)], colb)
        pltpu.sync_copy(w2h.at[pl.ds(rrow0, SUBS)], wb)

        def wmloop(i, _):
            for l in range(8):
                sl = pl.ds(16 * l, 16)
                rv = rowb[i, sl]
                cv = colb[i, sl]
                wv = wb[i, sl]
                wmb[pl.ds(128 * i + 16 * l, 16)] = jnp.where(rv == cv, 0.0, wv)
            return 0
        lax.fori_loop(0, SUBS, wmloop, 0)

        def sub_body(k, _):
            def grp(g, _):
                wv = wmb[pl.ds(128 * k + 16 * g, 16)]
                for l in range(16):
                    g16[16 * g + l, pl.ds(0, 16)] = jnp.full((16,), wv[l])
                return 0
            lax.fori_loop(0, 8, grp, 0)
            pltpu.sync_copy(g16, acc.at[rowb.at[k]], add=True)
            return 0
        lax.fori_loop(0, SUBS, sub_body, 0)
        return 0
    lax.fori_loop(0, SUPER // 2, super_body, 0)

    plsc.subcore_barrier()
    for ch in range(5):
        r = r0 + 128 * ch
        pltpu.sync_copy(acc.at[pl.ds(r, 128)], out.at[c].at[pl.ds(r, 128)])


def _deg_call(row2, col2, w2):
    mesh = plsc.VectorSubcoreMesh(core_axis_name="c", subcore_axis_name="s")
    return pl.kernel(
        _deg_body,
        out_type=jax.ShapeDtypeStruct((2, NP, 16), jnp.float32),
        mesh=mesh,
        compiler_params=pltpu.CompilerParams(use_tc_tiling_on_sc=False),
        scratch_types=[
            pltpu.VMEM((SUBS, 128), jnp.int32),
            pltpu.VMEM((SUBS, 128), jnp.int32),
            pltpu.VMEM((SUBS, 128), jnp.float32),
            pltpu.VMEM((2048,), jnp.float32),
            pltpu.VMEM((128, 16), jnp.float32),
            pltpu.VMEM((128, 16), jnp.float32),
            pltpu.VMEM_SHARED((NP, 16), jnp.float32),
        ],
    )(row2, col2, w2)


def _layer_body(Fh, ys, d2x, row2h, col2h, w2h, p_out, u_scr,
                rowb, colb, wb, wmall, g0, g1, g2, g3, yb, pb, db, ub, zb,
                acc, sg0, sg1, sg2, sg3, ss0, ss1, ss2, ss3):
    gbufs = (g0, g1, g2, g3)
    sgs = (sg0, sg1, sg2, sg3)
    sss = (ss0, ss1, ss2, ss3)
    c = lax.axis_index("c")
    s = lax.axis_index("s")
    r0 = s * ROWS_PER_TILE
    nb = Fh // 16
    _zero_2d(zb, 128, Fh)
    for ch in range(5):
        pltpu.sync_copy(zb, acc.at[pl.ds(r0 + 128 * ch, 128)])
    pltpu.sync_copy(d2x.at[pl.ds(r0, ROWS_PER_TILE)], db)

    def pre(sc, _):
        rrow0 = s * CHUNKS_PER_TILE + sc * SUBS
        pltpu.sync_copy(row2h.at[pl.ds(rrow0, SUBS)], rowb)
        pltpu.sync_copy(col2h.at[pl.ds(rrow0, SUBS)], colb)
        pltpu.sync_copy(w2h.at[pl.ds(rrow0, SUBS)], wb)

        def wmloop(i, _):
            for l in range(8):
                sl = pl.ds(16 * l, 16)
                rv = rowb[i, sl]
                cv = colb[i, sl]
                wv = wb[i, sl]
                wmall[pl.ds(2048 * sc + 128 * i + 16 * l, 16)] = jnp.where(
                    rv == cv, 0.0, wv)
            return 0
        lax.fori_loop(0, SUBS, wmloop, 0)
        return 0
    lax.fori_loop(0, SUPER, pre, 0)
    plsc.subcore_barrier()

    def multiply(G, wm_base):
        def grp(g, _):
            wv = wmall[pl.ds(wm_base + 16 * g, 16)]
            for l in range(16):
                e = 16 * g + l
                for b in range(nb):
                    sl = pl.ds(16 * b, 16)
                    G[e, sl] = G[e, sl] * wv[l]
            return 0
        lax.fori_loop(0, 8, grp, 0)

    for step in range(4):
        src = ys.at[c, 4] if step == 0 else u_scr.at[c]

        def super_body(sc, _):
            rrow0 = s * CHUNKS_PER_TILE + sc * SUBS
            pltpu.sync_copy(row2h.at[pl.ds(rrow0, SUBS)], rowb)
            pltpu.sync_copy(col2h.at[pl.ds(rrow0, SUBS)], colb)

            def quad(q, _):
                base = 4 * q
                ghs = [pltpu.async_copy(src.at[colb.at[base + b]],
                                        gbufs[b], sgs[b]) for b in range(4)]
                shs = []
                for b in range(4):
                    ghs[b].wait()
                    multiply(gbufs[b], 2048 * sc + 128 * (base + b))
                    shs.append(pltpu.async_copy(
                        gbufs[b], acc.at[rowb.at[base + b]], sss[b],
                        add=True))
                for h in shs:
                    h.wait()
                return 0
            lax.fori_loop(0, SUBS // 4, quad, 0)
            return 0
        lax.fori_loop(0, SUPER, super_body, 0)
        plsc.subcore_barrier()

        if step < 3:
            j = 3 - step

            def glue(chn, _):
                r = r0 + 128 * chn
                pltpu.sync_copy(acc.at[pl.ds(r, 128)], pb)
                pltpu.sync_copy(zb, acc.at[pl.ds(r, 128)])
                pltpu.sync_copy(ys.at[c, j].at[pl.ds(r, 128)], yb)

                def rowfn(i, _):
                    for b in range(nb):
                        sl = pl.ds(16 * b, 16)
                        ub[i, sl] = (yb[i, sl]
                                     - db[128 * chn + i, sl] * pb[i, sl])
                    return 0
                lax.fori_loop(0, 128, rowfn, 0)
                pltpu.sync_copy(ub, u_scr.at[c].at[pl.ds(r, 128)])
                return 0
            lax.fori_loop(0, 5, glue, 0)
            plsc.subcore_barrier()
        else:
            for ch in range(5):
                r = r0 + 128 * ch
                pltpu.sync_copy(acc.at[pl.ds(r, 128)],
                                p_out.at[c].at[pl.ds(r, 128)])


def _layer_call(Fh, ys, d2x, row2, col2, w2):
    mesh = plsc.VectorSubcoreMesh(core_axis_name="c", subcore_axis_name="s")
    fb = lambda shape: pltpu.VMEM(shape, jnp.float32)
    return pl.kernel(
        functools.partial(_layer_body, Fh),
        out_type=(jax.ShapeDtypeStruct((2, NP, Fh), jnp.float32),
                  jax.ShapeDtypeStruct((2, NP, Fh), jnp.float32)),
        mesh=mesh,
        compiler_params=pltpu.CompilerParams(use_tc_tiling_on_sc=False),
        scratch_types=[
            pltpu.VMEM((SUBS, 128), jnp.int32),
            pltpu.VMEM((SUBS, 128), jnp.int32),
            pltpu.VMEM((SUBS, 128), jnp.float32),
            pltpu.VMEM((EP // 16,), jnp.float32),
            fb((128, Fh)), fb((128, Fh)),
            fb((128, Fh)), fb((128, Fh)),
            fb((128, Fh)), fb((128, Fh)),
            fb((ROWS_PER_TILE, Fh)),
            fb((128, Fh)),
            fb((128, Fh)),
            pltpu.VMEM_SHARED((NP, Fh), jnp.float32),
        ] + [pltpu.SemaphoreType.DMA] * 8,
    )(ys, d2x, row2, col2, w2)


def _prep_body(x_ref, a1_ref, deg_ref, ys_ref, d232_ref, d216_ref, dinv_ref):
    deg = deg_ref[0, :, 0] + deg_ref[1, :, 0]
    safe = jnp.where(deg > 0, deg, 1.0)
    dinv = jnp.where(deg > 0, lax.rsqrt(safe), 0.0)
    d2 = dinv * dinv
    y = jnp.dot(x_ref[...], a1_ref[...], preferred_element_type=jnp.float32)
    for c in range(2):
        for j in range(5):
            blk = y[:, c * 160 + j * 32:c * 160 + (j + 1) * 32]
            if j > 0:
                blk = dinv[:, None] * blk
            ys_ref[c, j, :, :] = blk
    d232_ref[...] = jnp.broadcast_to(d2[:, None], d232_ref.shape)
    d216_ref[...] = jnp.broadcast_to(d2[:, None], d216_ref.shape)
    dinv_ref[...] = jnp.broadcast_to(dinv[:, None], dinv_ref.shape)


def _prep_call(x_pad, a1, deg16):
    R = 512
    grid = (NP // R,)
    return pl.pallas_call(
        _prep_body,
        grid=grid,
        in_specs=[
            pl.BlockSpec((R, 128), lambda i: (i, 0)),
            pl.BlockSpec((128, 320), lambda i: (0, 0)),
            pl.BlockSpec((2, R, 16), lambda i: (0, i, 0)),
        ],
        out_specs=[
            pl.BlockSpec((2, 5, R, 32), lambda i: (0, 0, i, 0)),
            pl.BlockSpec((R, 32), lambda i: (i, 0)),
            pl.BlockSpec((R, 16), lambda i: (i, 0)),
            pl.BlockSpec((R, 8), lambda i: (i, 0)),
        ],
        out_shape=[
            jax.ShapeDtypeStruct((2, 5, NP, 32), jnp.float32),
            jax.ShapeDtypeStruct((NP, 32), jnp.float32),
            jax.ShapeDtypeStruct((NP, 16), jnp.float32),
            jax.ShapeDtypeStruct((NP, 8), jnp.float32),
        ],
    )(x_pad, a1, deg16)


def _mid_body(p_ref, y0_ref, dinv_ref, a2_ref, bz_ref, bh_ref, ys2_ref):
    dinv = dinv_ref[:, 0]
    z = y0_ref[0, 0] - dinv[:, None] * p_ref[0] + bz_ref[...]
    h = y0_ref[1, 0] - dinv[:, None] * p_ref[1] + bh_ref[...]
    h1 = jax.nn.relu((1.0 - jax.nn.sigmoid(z)) * jnp.tanh(h))
    y2 = jnp.dot(h1, a2_ref[...], preferred_element_type=jnp.float32)
    for c in range(2):
        for j in range(5):
            blk = y2[:, c * 80 + j * 16:c * 80 + (j + 1) * 16]
            if j > 0:
                blk = dinv[:, None] * blk
            ys2_ref[c, j, :, :] = blk


def _mid_call(p1, ys1, dinv8, a2, bz, bh):
    R = 512
    return pl.pallas_call(
        _mid_body,
        grid=(NP // R,),
        in_specs=[
            pl.BlockSpec((2, R, 32), lambda i: (0, i, 0)),
            pl.BlockSpec((2, 1, R, 32), lambda i: (0, 0, i, 0)),
            pl.BlockSpec((R, 8), lambda i: (i, 0)),
            pl.BlockSpec((32, 160), lambda i: (0, 0)),
            pl.BlockSpec((1, 32), lambda i: (0, 0)),
            pl.BlockSpec((1, 32), lambda i: (0, 0)),
        ],
        out_specs=pl.BlockSpec((2, 5, R, 16), lambda i: (0, 0, i, 0)),
        out_shape=jax.ShapeDtypeStruct((2, 5, NP, 16), jnp.float32),
    )(p1, ys1, dinv8, a2, bz, bh)


def _final_body(p_ref, y0_ref, dinv_ref, wl_ref, bl_ref, bz_ref, bh_ref,
                out_ref):
    dinv = dinv_ref[:, 0]
    z = y0_ref[0, 0] - dinv[:, None] * p_ref[0] + bz_ref[...]
    h = y0_ref[1, 0] - dinv[:, None] * p_ref[1] + bh_ref[...]
    h2 = jax.nn.relu((1.0 - jax.nn.sigmoid(z)) * jnp.tanh(h))
    logits = jnp.dot(h2, wl_ref[...],
                     preferred_element_type=jnp.float32) + bl_ref[...]
    m = jnp.max(logits, axis=-1, keepdims=True)
    e = jnp.exp(logits - m)
    ssum = jnp.sum(e, axis=-1, keepdims=True)
    out_ref[...] = logits - m - jnp.log(ssum)


def _final_call(p2, ys2, dinv8, wlp, blp, bz, bh):
    R = 512
    return pl.pallas_call(
        _final_body,
        grid=(NP // R,),
        in_specs=[
            pl.BlockSpec((2, R, 16), lambda i: (0, i, 0)),
            pl.BlockSpec((2, 1, R, 16), lambda i: (0, 0, i, 0)),
            pl.BlockSpec((R, 8), lambda i: (i, 0)),
            pl.BlockSpec((16, 128), lambda i: (0, 0)),
            pl.BlockSpec((1, 128), lambda i: (0, 0)),
            pl.BlockSpec((1, 16), lambda i: (0, 0)),
            pl.BlockSpec((1, 16), lambda i: (0, 0)),
        ],
        out_specs=pl.BlockSpec((R, 128), lambda i: (i, 0)),
        out_shape=jax.ShapeDtypeStruct((NP, 128), jnp.float32),
    )(p2, ys2, dinv8, wlp, blp, bz, bh)


def kernel(x, edge_weight, Wxz1, bxz1, Whz1, bhz1, Wxr1, bxr1, Whr1, bhr1,
           Wxh1, bxh1, Whh1, bhh1, Wxz2, bxz2, Whz2, bhz2, Wxr2, bxr2,
           Whr2, bhr2, Wxh2, bxh2, Whh2, bhh2, Wl, bl, edge_index):
    row = edge_index[0]
    col = edge_index[1]
    row2 = jnp.concatenate(
        [row, jnp.zeros((EP - E,), jnp.int32)]).reshape(EP // 128, 128)
    col2 = jnp.concatenate(
        [col, jnp.zeros((EP - E,), jnp.int32)]).reshape(EP // 128, 128)
    w2 = jnp.concatenate(
        [edge_weight, jnp.zeros((EP - E,), jnp.float32)]).reshape(
            EP // 128, 128)
    x_pad = jnp.pad(x, ((0, NP - N), (0, 0)))

    az1 = _power_weights(Wxz1)
    ah1 = _power_weights(Wxh1)
    a1 = jnp.concatenate([a for pair in (az1, ah1) for a in pair], axis=1)
    az2 = _power_weights(Wxz2)
    ah2 = _power_weights(Wxh2)
    a2 = jnp.concatenate([a for pair in (az2, ah2) for a in pair], axis=1)
    bz1 = (bxz1 + bhz1).reshape(1, 32)
    bh1 = (bxh1 + bhh1).reshape(1, 32)
    bz2 = (bxz2 + bhz2).reshape(1, 16)
    bh2 = (bxh2 + bhh2).reshape(1, 16)
    wlp = jnp.pad(Wl.T, ((0, 0), (0, 118)))
    blp = jnp.pad(bl, (0, 118), constant_values=-1e30).reshape(1, 128)

    deg16 = _deg_call(row2, col2, w2)
    ys1, d232, d216, dinv8 = _prep_call(x_pad, a1, deg16)
    p1, _ = _layer_call(32, ys1, d232, row2, col2, w2)
    ys2 = _mid_call(p1, ys1, dinv8, a2, bz1, bh1)
    p2, _ = _layer_call(16, ys2, d216, row2, col2, w2)
    outp = _final_call(p2, ys2, dinv8, wlp, blp, bz2, bh2)
    return outp[:N, :10]


# speedup vs baseline: 19.0396x; 1.0650x over previous
import functools

import jax
import jax.numpy as jnp
from jax import lax
from jax.experimental import pallas as pl
from jax.experimental.pallas import tpu as pltpu
from jax.experimental.pallas import tpu_sc as plsc

N = 10000
E = 320000
NP = 10240
EP = 327680
ROWS_PER_TILE = NP // 16
CHUNKS_PER_TILE = EP // 16 // 128
SUPER = 10
SUBS = 16


def _power_weights(W):
    A0 = W[0] - W[2] + W[4]
    A1 = W[1] - 3.0 * W[3]
    A2 = 2.0 * W[2] - 8.0 * W[4]
    A3 = 4.0 * W[3]
    A4 = 8.0 * W[4]
    return [A0, A1, A2, A3, A4]


def _zeros16():
    return jnp.zeros((16,), jnp.float32)


def _zero_2d(ref, nrows, fcols):
    def body(i, _):
        for b in range(fcols // 16):
            ref[i, pl.ds(16 * b, 16)] = _zeros16()
        return 0
    lax.fori_loop(0, nrows, body, 0)


def _deg_body(row2h, col2h, w2h, out, rowb, colb, wb, wmb, g16, zb, acc):
    c = lax.axis_index("c")
    s = lax.axis_index("s")
    r0 = s * ROWS_PER_TILE
    _zero_2d(g16, 128, 16)
    _zero_2d(zb, 128, 16)
    for ch in range(5):
        pltpu.sync_copy(zb, acc.at[pl.ds(r0 + 128 * ch, 128)])
    plsc.subcore_barrier()

    wid = c * 16 + s

    def super_body(sc, _):
        rrow0 = wid * (CHUNKS_PER_TILE // 2) + sc * SUBS
        pltpu.sync_copy(row2h.at[pl.ds(rrow0, SUBS)], rowb)
        pltpu.sync_copy(col2h.at[pl.ds(rrow0, SUBS)], colb)
        pltpu.sync_copy(w2h.at[pl.ds(rrow0, SUBS)], wb)

        def wmloop(i, _):
            for l in range(8):
                sl = pl.ds(16 * l, 16)
                rv = rowb[i, sl]
                cv = colb[i, sl]
                wv = wb[i, sl]
                wmb[pl.ds(128 * i + 16 * l, 16)] = jnp.where(rv == cv, 0.0, wv)
            return 0
        lax.fori_loop(0, SUBS, wmloop, 0)

        def sub_body(k, _):
            def grp(g, _):
                wv = wmb[pl.ds(128 * k + 16 * g, 16)]
                for l in range(16):
                    g16[16 * g + l, pl.ds(0, 16)] = jnp.full((16,), wv[l])
                return 0
            lax.fori_loop(0, 8, grp, 0)
            pltpu.sync_copy(g16, acc.at[rowb.at[k]], add=True)
            return 0
        lax.fori_loop(0, SUBS, sub_body, 0)
        return 0
    lax.fori_loop(0, SUPER // 2, super_body, 0)

    plsc.subcore_barrier()
    for ch in range(5):
        r = r0 + 128 * ch
        pltpu.sync_copy(acc.at[pl.ds(r, 128)], out.at[c].at[pl.ds(r, 128)])


def _deg_call(row2, col2, w2):
    mesh = plsc.VectorSubcoreMesh(core_axis_name="c", subcore_axis_name="s")
    return pl.kernel(
        _deg_body,
        out_type=jax.ShapeDtypeStruct((2, NP, 16), jnp.float32),
        mesh=mesh,
        compiler_params=pltpu.CompilerParams(use_tc_tiling_on_sc=False),
        scratch_types=[
            pltpu.VMEM((SUBS, 128), jnp.int32),
            pltpu.VMEM((SUBS, 128), jnp.int32),
            pltpu.VMEM((SUBS, 128), jnp.float32),
            pltpu.VMEM((2048,), jnp.float32),
            pltpu.VMEM((128, 16), jnp.float32),
            pltpu.VMEM((128, 16), jnp.float32),
            pltpu.VMEM_SHARED((NP, 16), jnp.float32),
        ],
    )(row2, col2, w2)


def _layer_body(Fh, ys, d2x, row2h, col2h, w2h, p_out, u_scr,
                row_all, col_all, wb, wmall, g0, g1, g2, g3, yb, pb, db, ub,
                zb, acc, sg0, sg1, sg2, sg3, ss0, ss1, ss2, ss3):
    gbufs = (g0, g1, g2, g3)
    sgs = (sg0, sg1, sg2, sg3)
    sss = (ss0, ss1, ss2, ss3)
    c = lax.axis_index("c")
    s = lax.axis_index("s")
    r0 = s * ROWS_PER_TILE
    nb = Fh // 16
    _zero_2d(zb, 128, Fh)
    for ch in range(5):
        pltpu.sync_copy(zb, acc.at[pl.ds(r0 + 128 * ch, 128)])
    pltpu.sync_copy(d2x.at[pl.ds(r0, ROWS_PER_TILE)], db)

    def pre(sc, _):
        rrow0 = s * CHUNKS_PER_TILE + sc * SUBS
        pltpu.sync_copy(row2h.at[pl.ds(rrow0, SUBS)], row_all.at[pl.ds(0, SUBS)])
        pltpu.sync_copy(col2h.at[pl.ds(rrow0, SUBS)], col_all.at[pl.ds(0, SUBS)])
        pltpu.sync_copy(w2h.at[pl.ds(rrow0, SUBS)], wb)

        def wmloop(i, _):
            for l in range(8):
                sl = pl.ds(16 * l, 16)
                rv = row_all[i, sl]
                cv = col_all[i, sl]
                wv = wb[i, sl]
                wmall[pl.ds(2048 * sc + 128 * i + 16 * l, 16)] = jnp.where(
                    rv == cv, 0.0, wv)
            return 0
        lax.fori_loop(0, SUBS, wmloop, 0)
        return 0
    lax.fori_loop(0, SUPER, pre, 0)
    plsc.subcore_barrier()

    def multiply(G, wm_base):
        def grp(g, _):
            wv = wmall[pl.ds(wm_base + 16 * g, 16)]
            for l in range(16):
                e = 16 * g + l
                for b in range(nb):
                    sl = pl.ds(16 * b, 16)
                    G[e, sl] = G[e, sl] * wv[l]
            return 0
        lax.fori_loop(0, 8, grp, 0)

    for step in range(4):
        src = ys.at[c, 4] if step == 0 else u_scr.at[c]
        dummy = ys.at[c, 0].at[pl.ds(0, 128)]

        def super_body(sc, _):
            half = lax.rem(sc, 2) * SUBS
            rrow0 = s * CHUNKS_PER_TILE + sc * SUBS
            pltpu.sync_copy(row2h.at[pl.ds(rrow0, SUBS)],
                            row_all.at[pl.ds(half, SUBS)])
            pltpu.sync_copy(col2h.at[pl.ds(rrow0, SUBS)],
                            col_all.at[pl.ds(half, SUBS)])

            def quad(q, _):
                ghs = []
                for b in range(4):
                    @pl.when(jnp.logical_or(sc > 0, q > 0))
                    def _():
                        pltpu.make_async_copy(dummy, gbufs[b], sss[b]).wait()
                    ghs.append(pltpu.async_copy(
                        src.at[col_all.at[half + 4 * q + b]],
                        gbufs[b], sgs[b]))
                for b in range(4):
                    ghs[b].wait()
                    multiply(gbufs[b], 2048 * sc + 128 * (4 * q + b))
                    pltpu.async_copy(
                        gbufs[b], acc.at[row_all.at[half + 4 * q + b]],
                        sss[b], add=True)
                return 0
            lax.fori_loop(0, 4, quad, 0)
            return 0
        lax.fori_loop(0, SUPER, super_body, 0)
        for b in range(4):
            pltpu.make_async_copy(dummy, gbufs[b], sss[b]).wait()
        plsc.subcore_barrier()

        if step < 3:
            j = 3 - step

            def glue(chn, _):
                r = r0 + 128 * chn
                pltpu.sync_copy(acc.at[pl.ds(r, 128)], pb)
                pltpu.sync_copy(zb, acc.at[pl.ds(r, 128)])
                pltpu.sync_copy(ys.at[c, j].at[pl.ds(r, 128)], yb)

                def rowfn(i, _):
                    for b in range(nb):
                        sl = pl.ds(16 * b, 16)
                        ub[i, sl] = (yb[i, sl]
                                     - db[128 * chn + i, sl] * pb[i, sl])
                    return 0
                lax.fori_loop(0, 128, rowfn, 0)
                pltpu.sync_copy(ub, u_scr.at[c].at[pl.ds(r, 128)])
                return 0
            lax.fori_loop(0, 5, glue, 0)
            plsc.subcore_barrier()
        else:
            for ch in range(5):
                r = r0 + 128 * ch
                pltpu.sync_copy(acc.at[pl.ds(r, 128)],
                                p_out.at[c].at[pl.ds(r, 128)])


def _layer_call(Fh, ys, d2x, row2, col2, w2):
    mesh = plsc.VectorSubcoreMesh(core_axis_name="c", subcore_axis_name="s")
    fb = lambda shape: pltpu.VMEM(shape, jnp.float32)
    return pl.kernel(
        functools.partial(_layer_body, Fh),
        out_type=(jax.ShapeDtypeStruct((2, NP, Fh), jnp.float32),
                  jax.ShapeDtypeStruct((2, NP, Fh), jnp.float32)),
        mesh=mesh,
        compiler_params=pltpu.CompilerParams(use_tc_tiling_on_sc=False),
        scratch_types=[
            pltpu.VMEM((2 * SUBS, 128), jnp.int32),
            pltpu.VMEM((2 * SUBS, 128), jnp.int32),
            pltpu.VMEM((SUBS, 128), jnp.float32),
            pltpu.VMEM((EP // 16,), jnp.float32),
            fb((128, Fh)), fb((128, Fh)),
            fb((128, Fh)), fb((128, Fh)),
            fb((128, Fh)), fb((128, Fh)),
            fb((ROWS_PER_TILE, Fh)),
            fb((128, Fh)),
            fb((128, Fh)),
            pltpu.VMEM_SHARED((NP, Fh), jnp.float32),
        ] + [pltpu.SemaphoreType.DMA] * 8,
    )(ys, d2x, row2, col2, w2)


def _prep_body(x_ref, a1_ref, deg_ref, ys_ref, d232_ref, d216_ref, dinv_ref):
    deg = deg_ref[0, :, 0] + deg_ref[1, :, 0]
    safe = jnp.where(deg > 0, deg, 1.0)
    dinv = jnp.where(deg > 0, lax.rsqrt(safe), 0.0)
    d2 = dinv * dinv
    y = jnp.dot(x_ref[...], a1_ref[...], preferred_element_type=jnp.float32)
    for c in range(2):
        for j in range(5):
            blk = y[:, c * 160 + j * 32:c * 160 + (j + 1) * 32]
            if j > 0:
                blk = dinv[:, None] * blk
            ys_ref[c, j, :, :] = blk
    d232_ref[...] = jnp.broadcast_to(d2[:, None], d232_ref.shape)
    d216_ref[...] = jnp.broadcast_to(d2[:, None], d216_ref.shape)
    dinv_ref[...] = jnp.broadcast_to(dinv[:, None], dinv_ref.shape)


def _prep_call(x_pad, a1, deg16):
    R = 512
    grid = (NP // R,)
    return pl.pallas_call(
        _prep_body,
        grid=grid,
        in_specs=[
            pl.BlockSpec((R, 128), lambda i: (i, 0)),
            pl.BlockSpec((128, 320), lambda i: (0, 0)),
            pl.BlockSpec((2, R, 16), lambda i: (0, i, 0)),
        ],
        out_specs=[
            pl.BlockSpec((2, 5, R, 32), lambda i: (0, 0, i, 0)),
            pl.BlockSpec((R, 32), lambda i: (i, 0)),
            pl.BlockSpec((R, 16), lambda i: (i, 0)),
            pl.BlockSpec((R, 8), lambda i: (i, 0)),
        ],
        out_shape=[
            jax.ShapeDtypeStruct((2, 5, NP, 32), jnp.float32),
            jax.ShapeDtypeStruct((NP, 32), jnp.float32),
            jax.ShapeDtypeStruct((NP, 16), jnp.float32),
            jax.ShapeDtypeStruct((NP, 8), jnp.float32),
        ],
    )(x_pad, a1, deg16)


def _mid_body(p_ref, y0_ref, dinv_ref, a2_ref, bz_ref, bh_ref, ys2_ref):
    dinv = dinv_ref[:, 0]
    z = y0_ref[0, 0] - dinv[:, None] * p_ref[0] + bz_ref[...]
    h = y0_ref[1, 0] - dinv[:, None] * p_ref[1] + bh_ref[...]
    h1 = jax.nn.relu((1.0 - jax.nn.sigmoid(z)) * jnp.tanh(h))
    y2 = jnp.dot(h1, a2_ref[...], preferred_element_type=jnp.float32)
    for c in range(2):
        for j in range(5):
            blk = y2[:, c * 80 + j * 16:c * 80 + (j + 1) * 16]
            if j > 0:
                blk = dinv[:, None] * blk
            ys2_ref[c, j, :, :] = blk


def _mid_call(p1, ys1, dinv8, a2, bz, bh):
    R = 512
    return pl.pallas_call(
        _mid_body,
        grid=(NP // R,),
        in_specs=[
            pl.BlockSpec((2, R, 32), lambda i: (0, i, 0)),
            pl.BlockSpec((2, 1, R, 32), lambda i: (0, 0, i, 0)),
            pl.BlockSpec((R, 8), lambda i: (i, 0)),
            pl.BlockSpec((32, 160), lambda i: (0, 0)),
            pl.BlockSpec((1, 32), lambda i: (0, 0)),
            pl.BlockSpec((1, 32), lambda i: (0, 0)),
        ],
        out_specs=pl.BlockSpec((2, 5, R, 16), lambda i: (0, 0, i, 0)),
        out_shape=jax.ShapeDtypeStruct((2, 5, NP, 16), jnp.float32),
    )(p1, ys1, dinv8, a2, bz, bh)


def _final_body(p_ref, y0_ref, dinv_ref, wl_ref, bl_ref, bz_ref, bh_ref,
                out_ref):
    dinv = dinv_ref[:, 0]
    z = y0_ref[0, 0] - dinv[:, None] * p_ref[0] + bz_ref[...]
    h = y0_ref[1, 0] - dinv[:, None] * p_ref[1] + bh_ref[...]
    h2 = jax.nn.relu((1.0 - jax.nn.sigmoid(z)) * jnp.tanh(h))
    logits = jnp.dot(h2, wl_ref[...],
                     preferred_element_type=jnp.float32) + bl_ref[...]
    m = jnp.max(logits, axis=-1, keepdims=True)
    e = jnp.exp(logits - m)
    ssum = jnp.sum(e, axis=-1, keepdims=True)
    out_ref[...] = logits - m - jnp.log(ssum)


def _final_call(p2, ys2, dinv8, wlp, blp, bz, bh):
    R = 512
    return pl.pallas_call(
        _final_body,
        grid=(NP // R,),
        in_specs=[
            pl.BlockSpec((2, R, 16), lambda i: (0, i, 0)),
            pl.BlockSpec((2, 1, R, 16), lambda i: (0, 0, i, 0)),
            pl.BlockSpec((R, 8), lambda i: (i, 0)),
            pl.BlockSpec((16, 128), lambda i: (0, 0)),
            pl.BlockSpec((1, 128), lambda i: (0, 0)),
            pl.BlockSpec((1, 16), lambda i: (0, 0)),
            pl.BlockSpec((1, 16), lambda i: (0, 0)),
        ],
        out_specs=pl.BlockSpec((R, 128), lambda i: (i, 0)),
        out_shape=jax.ShapeDtypeStruct((NP, 128), jnp.float32),
    )(p2, ys2, dinv8, wlp, blp, bz, bh)


def kernel(x, edge_weight, Wxz1, bxz1, Whz1, bhz1, Wxr1, bxr1, Whr1, bhr1,
           Wxh1, bxh1, Whh1, bhh1, Wxz2, bxz2, Whz2, bhz2, Wxr2, bxr2,
           Whr2, bhr2, Wxh2, bxh2, Whh2, bhh2, Wl, bl, edge_index):
    row = edge_index[0]
    col = edge_index[1]
    row2 = jnp.concatenate(
        [row, jnp.zeros((EP - E,), jnp.int32)]).reshape(EP // 128, 128)
    col2 = jnp.concatenate(
        [col, jnp.zeros((EP - E,), jnp.int32)]).reshape(EP // 128, 128)
    w2 = jnp.concatenate(
        [edge_weight, jnp.zeros((EP - E,), jnp.float32)]).reshape(
            EP // 128, 128)
    x_pad = jnp.pad(x, ((0, NP - N), (0, 0)))

    az1 = _power_weights(Wxz1)
    ah1 = _power_weights(Wxh1)
    a1 = jnp.concatenate([a for pair in (az1, ah1) for a in pair], axis=1)
    az2 = _power_weights(Wxz2)
    ah2 = _power_weights(Wxh2)
    a2 = jnp.concatenate([a for pair in (az2, ah2) for a in pair], axis=1)
    bz1 = (bxz1 + bhz1).reshape(1, 32)
    bh1 = (bxh1 + bhh1).reshape(1, 32)
    bz2 = (bxz2 + bhz2).reshape(1, 16)
    bh2 = (bxh2 + bhh2).reshape(1, 16)
    wlp = jnp.pad(Wl.T, ((0, 0), (0, 118)))
    blp = jnp.pad(bl, (0, 118), constant_values=-1e30).reshape(1, 128)

    deg16 = _deg_call(row2, col2, w2)
    ys1, d232, d216, dinv8 = _prep_call(x_pad, a1, deg16)
    p1, _ = _layer_call(32, ys1, d232, row2, col2, w2)
    ys2 = _mid_call(p1, ys1, dinv8, a2, bz1, bh1)
    p2, _ = _layer_call(16, ys2, d216, row2, col2, w2)
    outp = _final_call(p2, ys2, dinv8, wlp, blp, bz2, bh2)
    return outp[:N, :10]


# speedup vs baseline: 28.4259x; 1.4930x over previous
import functools

import jax
import jax.numpy as jnp
from jax import lax
from jax.experimental import pallas as pl
from jax.experimental.pallas import tpu as pltpu
from jax.experimental.pallas import tpu_sc as plsc

N = 10000
E = 320000
NP = 10240
EP = 327680
ROWS_PER_TILE = NP // 16
CHUNKS_PER_TILE = EP // 16 // 128
SUPER = 10
SUBS = 16


def _power_weights(W):
    A0 = W[0] - W[2] + W[4]
    A1 = W[1] - 3.0 * W[3]
    A2 = 2.0 * W[2] - 8.0 * W[4]
    A3 = 4.0 * W[3]
    A4 = 8.0 * W[4]
    return [A0, A1, A2, A3, A4]


def _zeros16():
    return jnp.zeros((16,), jnp.float32)


def _zero_2d(ref, nrows, fcols):
    def body(i, _):
        for b in range(fcols // 16):
            ref[i, pl.ds(16 * b, 16)] = _zeros16()
        return 0
    lax.fori_loop(0, nrows, body, 0)


def _deg_body(row2h, col2h, w2h, out, rowb, colb, wb, wmb, g16, zb, acc):
    c = lax.axis_index("c")
    s = lax.axis_index("s")
    r0 = s * ROWS_PER_TILE
    _zero_2d(g16, 128, 16)
    _zero_2d(zb, 128, 16)
    for ch in range(5):
        pltpu.sync_copy(zb, acc.at[pl.ds(r0 + 128 * ch, 128)])
    plsc.subcore_barrier()

    wid = c * 16 + s

    def super_body(sc, _):
        rrow0 = wid * (CHUNKS_PER_TILE // 2) + sc * SUBS
        pltpu.sync_copy(row2h.at[pl.ds(rrow0, SUBS)], rowb)
        pltpu.sync_copy(col2h.at[pl.ds(rrow0, SUBS)], colb)
        pltpu.sync_copy(w2h.at[pl.ds(rrow0, SUBS)], wb)

        def wmloop(i, _):
            for l in range(8):
                sl = pl.ds(16 * l, 16)
                rv = rowb[i, sl]
                cv = colb[i, sl]
                wv = wb[i, sl]
                wmb[pl.ds(128 * i + 16 * l, 16)] = jnp.where(rv == cv, 0.0, wv)
            return 0
        lax.fori_loop(0, SUBS, wmloop, 0)

        def sub_body(k, _):
            def grp(g, _):
                wv = wmb[pl.ds(128 * k + 16 * g, 16)]
                for l in range(16):
                    g16[16 * g + l, pl.ds(0, 16)] = jnp.full((16,), wv[l])
                return 0
            lax.fori_loop(0, 8, grp, 0)
            pltpu.sync_copy(g16, acc.at[rowb.at[k]], add=True)
            return 0
        lax.fori_loop(0, SUBS, sub_body, 0)
        return 0
    lax.fori_loop(0, SUPER // 2, super_body, 0)

    plsc.subcore_barrier()
    for ch in range(5):
        r = r0 + 128 * ch
        pltpu.sync_copy(acc.at[pl.ds(r, 128)], out.at[c].at[pl.ds(r, 128)])


def _deg_call(row2, col2, w2):
    mesh = plsc.VectorSubcoreMesh(core_axis_name="c", subcore_axis_name="s")
    return pl.kernel(
        _deg_body,
        out_type=jax.ShapeDtypeStruct((2, NP, 16), jnp.float32),
        mesh=mesh,
        compiler_params=pltpu.CompilerParams(use_tc_tiling_on_sc=False),
        scratch_types=[
            pltpu.VMEM((SUBS, 128), jnp.int32),
            pltpu.VMEM((SUBS, 128), jnp.int32),
            pltpu.VMEM((SUBS, 128), jnp.float32),
            pltpu.VMEM((2048,), jnp.float32),
            pltpu.VMEM((128, 16), jnp.float32),
            pltpu.VMEM((128, 16), jnp.float32),
            pltpu.VMEM_SHARED((NP, 16), jnp.float32),
        ],
    )(row2, col2, w2)


def _layer_body(Fh, ys, d2x, row2h, col2h, w2h, p_out,
                row_all, col_all, wb, wmall, g0, g1, g2, g3, yb, pb, db, ub,
                zb, acc, u_sp, sg0, sg1, sg2, sg3, ss0, ss1, ss2, ss3):
    gbufs = (g0, g1, g2, g3)
    sgs = (sg0, sg1, sg2, sg3)
    sss = (ss0, ss1, ss2, ss3)
    c = lax.axis_index("c")
    s = lax.axis_index("s")
    r0 = s * ROWS_PER_TILE
    nb = Fh // 16
    _zero_2d(zb, 128, Fh)
    for ch in range(5):
        pltpu.sync_copy(zb, acc.at[pl.ds(r0 + 128 * ch, 128)])
    pltpu.sync_copy(d2x.at[pl.ds(r0, ROWS_PER_TILE)], db)
    pltpu.sync_copy(ys.at[c, 4].at[pl.ds(r0, ROWS_PER_TILE)],
                    u_sp.at[pl.ds(r0, ROWS_PER_TILE)])

    def pre(sc, _):
        rrow0 = s * CHUNKS_PER_TILE + sc * SUBS
        pltpu.sync_copy(row2h.at[pl.ds(rrow0, SUBS)], row_all.at[pl.ds(0, SUBS)])
        pltpu.sync_copy(col2h.at[pl.ds(rrow0, SUBS)], col_all.at[pl.ds(0, SUBS)])
        pltpu.sync_copy(w2h.at[pl.ds(rrow0, SUBS)], wb)

        def wmloop(i, _):
            for l in range(8):
                sl = pl.ds(16 * l, 16)
                rv = row_all[i, sl]
                cv = col_all[i, sl]
                wv = wb[i, sl]
                wmall[pl.ds(2048 * sc + 128 * i + 16 * l, 16)] = jnp.where(
                    rv == cv, 0.0, wv)
            return 0
        lax.fori_loop(0, SUBS, wmloop, 0)
        return 0
    lax.fori_loop(0, SUPER, pre, 0)
    plsc.subcore_barrier()

    def multiply(G, wm_base):
        def grp(g, _):
            wv = wmall[pl.ds(wm_base + 16 * g, 16)]
            for l in range(16):
                e = 16 * g + l
                for b in range(nb):
                    sl = pl.ds(16 * b, 16)
                    G[e, sl] = G[e, sl] * wv[l]
            return 0
        lax.fori_loop(0, 8, grp, 0)

    for step in range(4):
        src = u_sp
        dummy = ys.at[c, 0].at[pl.ds(0, 128)]

        def super_body(sc, _):
            half = lax.rem(sc, 2) * SUBS
            rrow0 = s * CHUNKS_PER_TILE + sc * SUBS
            pltpu.sync_copy(row2h.at[pl.ds(rrow0, SUBS)],
                            row_all.at[pl.ds(half, SUBS)])
            pltpu.sync_copy(col2h.at[pl.ds(rrow0, SUBS)],
                            col_all.at[pl.ds(half, SUBS)])

            def quad(q, _):
                ghs = []
                for b in range(4):
                    @pl.when(jnp.logical_or(sc > 0, q > 0))
                    def _():
                        pltpu.make_async_copy(dummy, gbufs[b], sss[b]).wait()
                    ghs.append(pltpu.async_copy(
                        src.at[col_all.at[half + 4 * q + b]],
                        gbufs[b], sgs[b]))
                for b in range(4):
                    ghs[b].wait()
                    multiply(gbufs[b], 2048 * sc + 128 * (4 * q + b))
                    pltpu.async_copy(
                        gbufs[b], acc.at[row_all.at[half + 4 * q + b]],
                        sss[b], add=True)
                return 0
            lax.fori_loop(0, 4, quad, 0)
            return 0
        lax.fori_loop(0, SUPER, super_body, 0)
        for b in range(4):
            pltpu.make_async_copy(dummy, gbufs[b], sss[b]).wait()
        plsc.subcore_barrier()

        if step < 3:
            j = 3 - step

            def glue(chn, _):
                r = r0 + 128 * chn
                pltpu.sync_copy(acc.at[pl.ds(r, 128)], pb)
                pltpu.sync_copy(zb, acc.at[pl.ds(r, 128)])
                pltpu.sync_copy(ys.at[c, j].at[pl.ds(r, 128)], yb)

                def rowfn(i, _):
                    for b in range(nb):
                        sl = pl.ds(16 * b, 16)
                        ub[i, sl] = (yb[i, sl]
                                     - db[128 * chn + i, sl] * pb[i, sl])
                    return 0
                lax.fori_loop(0, 128, rowfn, 0)
                pltpu.sync_copy(ub, u_sp.at[pl.ds(r, 128)])
                return 0
            lax.fori_loop(0, 5, glue, 0)
            plsc.subcore_barrier()
        else:
            for ch in range(5):
                r = r0 + 128 * ch
                pltpu.sync_copy(acc.at[pl.ds(r, 128)],
                                p_out.at[c].at[pl.ds(r, 128)])


def _layer_call(Fh, ys, d2x, row2, col2, w2):
    mesh = plsc.VectorSubcoreMesh(core_axis_name="c", subcore_axis_name="s")
    fb = lambda shape: pltpu.VMEM(shape, jnp.float32)
    return pl.kernel(
        functools.partial(_layer_body, Fh),
        out_type=jax.ShapeDtypeStruct((2, NP, Fh), jnp.float32),
        mesh=mesh,
        compiler_params=pltpu.CompilerParams(use_tc_tiling_on_sc=False),
        scratch_types=[
            pltpu.VMEM((2 * SUBS, 128), jnp.int32),
            pltpu.VMEM((2 * SUBS, 128), jnp.int32),
            pltpu.VMEM((SUBS, 128), jnp.float32),
            pltpu.VMEM((EP // 16,), jnp.float32),
            fb((128, Fh)), fb((128, Fh)),
            fb((128, Fh)), fb((128, Fh)),
            fb((128, Fh)), fb((128, Fh)),
            fb((ROWS_PER_TILE, Fh)),
            fb((128, Fh)),
            fb((128, Fh)),
            pltpu.VMEM_SHARED((NP, Fh), jnp.float32),
            pltpu.VMEM_SHARED((NP, Fh), jnp.float32),
        ] + [pltpu.SemaphoreType.DMA] * 8,
    )(ys, d2x, row2, col2, w2)


def _prep_body(x_ref, a1_ref, deg_ref, ys_ref, d232_ref, d216_ref, dinv_ref):
    deg = deg_ref[0, :, 0] + deg_ref[1, :, 0]
    safe = jnp.where(deg > 0, deg, 1.0)
    dinv = jnp.where(deg > 0, lax.rsqrt(safe), 0.0)
    d2 = dinv * dinv
    y = jnp.dot(x_ref[...], a1_ref[...], preferred_element_type=jnp.float32)
    for c in range(2):
        for j in range(5):
            blk = y[:, c * 160 + j * 32:c * 160 + (j + 1) * 32]
            if j > 0:
                blk = dinv[:, None] * blk
            ys_ref[c, j, :, :] = blk
    d232_ref[...] = jnp.broadcast_to(d2[:, None], d232_ref.shape)
    d216_ref[...] = jnp.broadcast_to(d2[:, None], d216_ref.shape)
    dinv_ref[...] = jnp.broadcast_to(dinv[:, None], dinv_ref.shape)


def _prep_call(x_pad, a1, deg16):
    R = 512
    grid = (NP // R,)
    return pl.pallas_call(
        _prep_body,
        grid=grid,
        in_specs=[
            pl.BlockSpec((R, 128), lambda i: (i, 0)),
            pl.BlockSpec((128, 320), lambda i: (0, 0)),
            pl.BlockSpec((2, R, 16), lambda i: (0, i, 0)),
        ],
        out_specs=[
            pl.BlockSpec((2, 5, R, 32), lambda i: (0, 0, i, 0)),
            pl.BlockSpec((R, 32), lambda i: (i, 0)),
            pl.BlockSpec((R, 16), lambda i: (i, 0)),
            pl.BlockSpec((R, 8), lambda i: (i, 0)),
        ],
        out_shape=[
            jax.ShapeDtypeStruct((2, 5, NP, 32), jnp.float32),
            jax.ShapeDtypeStruct((NP, 32), jnp.float32),
            jax.ShapeDtypeStruct((NP, 16), jnp.float32),
            jax.ShapeDtypeStruct((NP, 8), jnp.float32),
        ],
    )(x_pad, a1, deg16)


def _mid_body(p_ref, y0_ref, dinv_ref, a2_ref, bz_ref, bh_ref, ys2_ref):
    dinv = dinv_ref[:, 0]
    z = y0_ref[0, 0] - dinv[:, None] * p_ref[0] + bz_ref[...]
    h = y0_ref[1, 0] - dinv[:, None] * p_ref[1] + bh_ref[...]
    h1 = jax.nn.relu((1.0 - jax.nn.sigmoid(z)) * jnp.tanh(h))
    y2 = jnp.dot(h1, a2_ref[...], preferred_element_type=jnp.float32)
    for c in range(2):
        for j in range(5):
            blk = y2[:, c * 80 + j * 16:c * 80 + (j + 1) * 16]
            if j > 0:
                blk = dinv[:, None] * blk
            ys2_ref[c, j, :, :] = blk


def _mid_call(p1, ys1, dinv8, a2, bz, bh):
    R = 512
    return pl.pallas_call(
        _mid_body,
        grid=(NP // R,),
        in_specs=[
            pl.BlockSpec((2, R, 32), lambda i: (0, i, 0)),
            pl.BlockSpec((2, 1, R, 32), lambda i: (0, 0, i, 0)),
            pl.BlockSpec((R, 8), lambda i: (i, 0)),
            pl.BlockSpec((32, 160), lambda i: (0, 0)),
            pl.BlockSpec((1, 32), lambda i: (0, 0)),
            pl.BlockSpec((1, 32), lambda i: (0, 0)),
        ],
        out_specs=pl.BlockSpec((2, 5, R, 16), lambda i: (0, 0, i, 0)),
        out_shape=jax.ShapeDtypeStruct((2, 5, NP, 16), jnp.float32),
    )(p1, ys1, dinv8, a2, bz, bh)


def _final_body(p_ref, y0_ref, dinv_ref, wl_ref, bl_ref, bz_ref, bh_ref,
                out_ref):
    dinv = dinv_ref[:, 0]
    z = y0_ref[0, 0] - dinv[:, None] * p_ref[0] + bz_ref[...]
    h = y0_ref[1, 0] - dinv[:, None] * p_ref[1] + bh_ref[...]
    h2 = jax.nn.relu((1.0 - jax.nn.sigmoid(z)) * jnp.tanh(h))
    logits = jnp.dot(h2, wl_ref[...],
                     preferred_element_type=jnp.float32) + bl_ref[...]
    m = jnp.max(logits, axis=-1, keepdims=True)
    e = jnp.exp(logits - m)
    ssum = jnp.sum(e, axis=-1, keepdims=True)
    out_ref[...] = logits - m - jnp.log(ssum)


def _final_call(p2, ys2, dinv8, wlp, blp, bz, bh):
    R = 512
    return pl.pallas_call(
        _final_body,
        grid=(NP // R,),
        in_specs=[
            pl.BlockSpec((2, R, 16), lambda i: (0, i, 0)),
            pl.BlockSpec((2, 1, R, 16), lambda i: (0, 0, i, 0)),
            pl.BlockSpec((R, 8), lambda i: (i, 0)),
            pl.BlockSpec((16, 128), lambda i: (0, 0)),
            pl.BlockSpec((1, 128), lambda i: (0, 0)),
            pl.BlockSpec((1, 16), lambda i: (0, 0)),
            pl.BlockSpec((1, 16), lambda i: (0, 0)),
        ],
        out_specs=pl.BlockSpec((R, 128), lambda i: (i, 0)),
        out_shape=jax.ShapeDtypeStruct((NP, 128), jnp.float32),
    )(p2, ys2, dinv8, wlp, blp, bz, bh)


def kernel(x, edge_weight, Wxz1, bxz1, Whz1, bhz1, Wxr1, bxr1, Whr1, bhr1,
           Wxh1, bxh1, Whh1, bhh1, Wxz2, bxz2, Whz2, bhz2, Wxr2, bxr2,
           Whr2, bhr2, Wxh2, bxh2, Whh2, bhh2, Wl, bl, edge_index):
    row = edge_index[0]
    col = edge_index[1]
    row2 = jnp.concatenate(
        [row, jnp.zeros((EP - E,), jnp.int32)]).reshape(EP // 128, 128)
    col2 = jnp.concatenate(
        [col, jnp.zeros((EP - E,), jnp.int32)]).reshape(EP // 128, 128)
    w2 = jnp.concatenate(
        [edge_weight, jnp.zeros((EP - E,), jnp.float32)]).reshape(
            EP // 128, 128)
    x_pad = jnp.pad(x, ((0, NP - N), (0, 0)))

    az1 = _power_weights(Wxz1)
    ah1 = _power_weights(Wxh1)
    a1 = jnp.concatenate([a for pair in (az1, ah1) for a in pair], axis=1)
    az2 = _power_weights(Wxz2)
    ah2 = _power_weights(Wxh2)
    a2 = jnp.concatenate([a for pair in (az2, ah2) for a in pair], axis=1)
    bz1 = (bxz1 + bhz1).reshape(1, 32)
    bh1 = (bxh1 + bhh1).reshape(1, 32)
    bz2 = (bxz2 + bhz2).reshape(1, 16)
    bh2 = (bxh2 + bhh2).reshape(1, 16)
    wlp = jnp.pad(Wl.T, ((0, 0), (0, 118)))
    blp = jnp.pad(bl, (0, 118), constant_values=-1e30).reshape(1, 128)

    deg16 = _deg_call(row2, col2, w2)
    ys1, d232, d216, dinv8 = _prep_call(x_pad, a1, deg16)
    p1 = _layer_call(32, ys1, d232, row2, col2, w2)
    ys2 = _mid_call(p1, ys1, dinv8, a2, bz1, bh1)
    p2 = _layer_call(16, ys2, d216, row2, col2, w2)
    outp = _final_call(p2, ys2, dinv8, wlp, blp, bz2, bh2)
    return outp[:N, :10]
